# baseline (speedup 1.0000x reference)
import functools

import jax
import jax.numpy as jnp
from jax import lax
from jax.experimental import pallas as pl
from jax.experimental.pallas import tpu as pltpu
from jax.experimental.pallas import tpu_sc as plsc

T = 2048
T2 = T * T
RW = T + 1
CH_ROWS = 16
CH = CH_ROWS * RW
NCHUNK = 128
LAST_CH = 15 * RW + 1
NG = 129
ACCW = NG * 16


def _sc_body(x_hbm, up_hbm, low_hbm, bufA, bufB, upacc, lowacc, semA, semB):
    w = lax.axis_index("s") * 2 + lax.axis_index("c")
    b = w // 8
    c0 = (w % 8) * 16
    base = b * T2
    lane = lax.iota(jnp.int32, 16)
    zero16 = jnp.zeros((16,), jnp.float32)

    def zbody(v, carry):
        upacc[pl.ds(v * 16, 16)] = zero16
        lowacc[pl.ds(v * 16, 16)] = zero16
        return carry

    lax.fori_loop(0, NG, zbody, 0)

    def start_full(buf, sem, c):
        pltpu.make_async_copy(
            x_hbm.at[pl.ds(base + c * CH, CH)], buf, sem
        ).start()

    def wait_full(buf, sem):
        pltpu.make_async_copy(x_hbm.at[pl.ds(0, CH)], buf, sem).wait()

    def start_odd(buf, sem, c):
        is_last = c == (NCHUNK - 1)

        @pl.when(jnp.logical_not(is_last))
        def _():
            start_full(buf, sem, c)

        @pl.when(is_last)
        def _():
            pltpu.make_async_copy(
                x_hbm.at[pl.ds(base + c * CH, LAST_CH)],
                buf.at[pl.ds(0, LAST_CH)],
                sem,
            ).start()

    def wait_odd(buf, sem, c):
        is_last = c == (NCHUNK - 1)

        @pl.when(jnp.logical_not(is_last))
        def _():
            wait_full(buf, sem)

        @pl.when(is_last)
        def _():
            pltpu.make_async_copy(
                x_hbm.at[pl.ds(0, LAST_CH)], buf.at[pl.ds(0, LAST_CH)], sem
            ).wait()

    def compute_chunk(buf, c):
        vS = (NCHUNK - 1) - c
        is_last = c == (NCHUNK - 1)
        notlast = jnp.where(is_last, 0.0, 1.0)

        def up_body(v, carry):
            acc = upacc[pl.ds(v * 16, 16)]
            for r in range(CH_ROWS):
                acc = acc + buf[pl.ds(r * RW + v * 16, 16)]
            upacc[pl.ds(v * 16, 16)] = acc
            return carry

        lax.fori_loop(0, vS, up_body, 0)

        ua = upacc[pl.ds(vS * 16, 16)]
        la = lowacc[pl.ds(vS * 16, 16)]
        for r in range(CH_ROWS):
            val = buf[pl.ds(r * RW + vS * 16, 16)]
            ua = ua + jnp.where(lane < (15 - r), val, 0.0)
            lval = jnp.where(lane > (15 - r), val, 0.0)
            if r >= 14:
                lval = lval * notlast
            la = la + lval
        upacc[pl.ds(vS * 16, 16)] = ua
        lowacc[pl.ds(vS * 16, 16)] = la

        def low_body(v, carry):
            acc = lowacc[pl.ds(v * 16, 16)]
            for r in range(CH_ROWS - 2):
                acc = acc + buf[pl.ds(r * RW + v * 16, 16)]
            tail = (
                buf[pl.ds(14 * RW + v * 16, 16)]
                + buf[pl.ds(15 * RW + v * 16, 16)]
            )
            acc = acc + tail * notlast
            lowacc[pl.ds(v * 16, 16)] = acc
            return carry

        lax.fori_loop(vS + 1, NG - 1, low_body, 0)

        tail = zero16
        for r in range(CH_ROWS - 2):
            tail = tail + buf[pl.ds((r + 1) * RW - 16, 16)]
        t2 = buf[pl.ds(15 * RW - 16, 16)] + buf[pl.ds(16 * RW - 16, 16)]
        tail = tail + t2 * notlast
        l128 = lowacc[pl.ds((NG - 1) * 16, 16)]
        lowacc[pl.ds((NG - 1) * 16, 16)] = l128 + jnp.where(lane == 15, tail, 0.0)

    start_full(bufA, semA, c0)

    def pair_body(p, carry):
        ce = c0 + 2 * p
        co = ce + 1
        wait_full(bufA, semA)
        start_odd(bufB, semB, co)
        compute_chunk(bufA, ce)
        wait_odd(bufB, semB, co)

        @pl.when(p < 7)
        def _():
            start_full(bufA, semA, ce + 2)

        compute_chunk(bufB, co)
        return carry

    lax.fori_loop(0, 8, pair_body, 0)

    pltpu.sync_copy(upacc, up_hbm.at[w])
    pltpu.sync_copy(lowacc, low_hbm.at[w])


def _finish_kernel(up_ref, low_ref, out_ref):
    rows = []
    for b in range(4):
        u = jnp.sum(up_ref[8 * b : 8 * b + 8, :], axis=0, keepdims=True)
        l = jnp.sum(low_ref[8 * b : 8 * b + 8, :], axis=0, keepdims=True)
        rows.append(
            jnp.concatenate(
                [l[:, 1025:2048], l[:, 2063:2064], u[:, 0:1024]], axis=1
            )
        )
    res = jnp.concatenate(rows, axis=0)
    jd = lax.broadcasted_iota(jnp.int32, (4, T), 1)
    count = (T - 1 - jnp.abs(jd - T // 2)).astype(jnp.float32)
    out_ref[...] = -res / count


@jax.jit
def kernel(x):
    B = x.shape[0]
    xf = x.reshape(-1)

    sc = pl.kernel(
        _sc_body,
        out_type=(
            jax.ShapeDtypeStruct((32, ACCW), jnp.float32),
            jax.ShapeDtypeStruct((32, ACCW), jnp.float32),
        ),
        mesh=plsc.VectorSubcoreMesh(core_axis_name="c", subcore_axis_name="s"),
        scratch_types=[
            pltpu.VMEM((CH,), jnp.float32),
            pltpu.VMEM((CH,), jnp.float32),
            pltpu.VMEM((ACCW,), jnp.float32),
            pltpu.VMEM((ACCW,), jnp.float32),
            pltpu.SemaphoreType.DMA,
            pltpu.SemaphoreType.DMA,
        ],
    )
    up, low = sc(xf)

    out = pl.pallas_call(
        _finish_kernel,
        out_shape=jax.ShapeDtypeStruct((B, T), jnp.float32),
    )(up, low)
    return out


# speedup vs baseline: 1.1071x; 1.1071x over previous
import functools

import jax
import jax.numpy as jnp
from jax import lax
from jax.experimental import pallas as pl
from jax.experimental.pallas import tpu as pltpu
from jax.experimental.pallas import tpu_sc as plsc

T = 2048
T2 = T * T
RW = T + 1
CH_ROWS = 16
CH = CH_ROWS * RW
NCHUNK = 128
LAST_CH = 15 * RW + 1
NG = 129
ACCW = NG * 16


def _sc_body(x_hbm, up_hbm, low_hbm, bufA, bufB, upacc, lowacc, semA, semB):
    w = lax.axis_index("s") * 2 + lax.axis_index("c")
    b = w // 8
    c0 = (w % 8) * 16
    base = b * T2
    lane = lax.iota(jnp.int32, 16)
    zero16 = jnp.zeros((16,), jnp.float32)

    def zbody(v, carry):
        upacc[pl.ds(v * 16, 16)] = zero16
        lowacc[pl.ds(v * 16, 16)] = zero16
        return carry

    lax.fori_loop(0, NG, zbody, 0)

    def start_full(buf, sem, c):
        pltpu.make_async_copy(
            x_hbm.at[pl.ds(base + c * CH, CH)], buf, sem
        ).start()

    def wait_full(buf, sem):
        pltpu.make_async_copy(x_hbm.at[pl.ds(0, CH)], buf, sem).wait()

    def start_odd(buf, sem, c):
        is_last = c == (NCHUNK - 1)

        @pl.when(jnp.logical_not(is_last))
        def _():
            start_full(buf, sem, c)

        @pl.when(is_last)
        def _():
            pltpu.make_async_copy(
                x_hbm.at[pl.ds(base + c * CH, LAST_CH)],
                buf.at[pl.ds(0, LAST_CH)],
                sem,
            ).start()

    def wait_odd(buf, sem, c):
        is_last = c == (NCHUNK - 1)

        @pl.when(jnp.logical_not(is_last))
        def _():
            wait_full(buf, sem)

        @pl.when(is_last)
        def _():
            pltpu.make_async_copy(
                x_hbm.at[pl.ds(0, LAST_CH)], buf.at[pl.ds(0, LAST_CH)], sem
            ).wait()

    def compute_chunk(buf, c):
        vS = (NCHUNK - 1) - c
        is_last = c == (NCHUNK - 1)
        notlast = jnp.where(is_last, 0.0, 1.0)

        def tree_sum(vals):
            while len(vals) > 1:
                nxt = [a + b for a, b in zip(vals[::2], vals[1::2])]
                if len(vals) % 2:
                    nxt[-1] = nxt[-1] + vals[-1]
                vals = nxt
            return vals[0]

        def up_body(v, carry):
            vals = [buf[pl.ds(r * RW + v * 16, 16)] for r in range(CH_ROWS)]
            upacc[pl.ds(v * 16, 16)] = upacc[pl.ds(v * 16, 16)] + tree_sum(vals)
            return carry

        lax.fori_loop(0, vS, up_body, 0)

        ua = upacc[pl.ds(vS * 16, 16)]
        la = lowacc[pl.ds(vS * 16, 16)]
        for r in range(CH_ROWS):
            val = buf[pl.ds(r * RW + vS * 16, 16)]
            ua = ua + jnp.where(lane < (15 - r), val, 0.0)
            lval = jnp.where(lane > (15 - r), val, 0.0)
            if r >= 14:
                lval = lval * notlast
            la = la + lval
        upacc[pl.ds(vS * 16, 16)] = ua
        lowacc[pl.ds(vS * 16, 16)] = la

        def low_body(v, carry):
            vals = [buf[pl.ds(r * RW + v * 16, 16)] for r in range(CH_ROWS - 2)]
            tail = (
                buf[pl.ds(14 * RW + v * 16, 16)]
                + buf[pl.ds(15 * RW + v * 16, 16)]
            )
            s = tree_sum(vals) + tail * notlast
            lowacc[pl.ds(v * 16, 16)] = lowacc[pl.ds(v * 16, 16)] + s
            return carry

        lax.fori_loop(vS + 1, NG - 1, low_body, 0)

        tvals = [buf[pl.ds((r + 1) * RW - 16, 16)] for r in range(CH_ROWS - 2)]
        t2 = buf[pl.ds(15 * RW - 16, 16)] + buf[pl.ds(16 * RW - 16, 16)]
        tail = tree_sum(tvals) + t2 * notlast
        l128 = lowacc[pl.ds((NG - 1) * 16, 16)]
        lowacc[pl.ds((NG - 1) * 16, 16)] = l128 + jnp.where(lane == 15, tail, 0.0)

    start_full(bufA, semA, c0)

    def pair_body(p, carry):
        ce = c0 + 2 * p
        co = ce + 1
        wait_full(bufA, semA)
        start_odd(bufB, semB, co)
        compute_chunk(bufA, ce)
        wait_odd(bufB, semB, co)

        @pl.when(p < 7)
        def _():
            start_full(bufA, semA, ce + 2)

        compute_chunk(bufB, co)
        return carry

    lax.fori_loop(0, 8, pair_body, 0)

    pltpu.sync_copy(upacc, up_hbm.at[w])
    pltpu.sync_copy(lowacc, low_hbm.at[w])


def _finish_kernel(up_ref, low_ref, out_ref):
    rows = []
    for b in range(4):
        u = jnp.sum(up_ref[8 * b : 8 * b + 8, :], axis=0, keepdims=True)
        l = jnp.sum(low_ref[8 * b : 8 * b + 8, :], axis=0, keepdims=True)
        rows.append(
            jnp.concatenate(
                [l[:, 1025:2048], l[:, 2063:2064], u[:, 0:1024]], axis=1
            )
        )
    res = jnp.concatenate(rows, axis=0)
    jd = lax.broadcasted_iota(jnp.int32, (4, T), 1)
    count = (T - 1 - jnp.abs(jd - T // 2)).astype(jnp.float32)
    out_ref[...] = -res / count


@jax.jit
def kernel(x):
    B = x.shape[0]
    xf = x.reshape(-1)

    sc = pl.kernel(
        _sc_body,
        out_type=(
            jax.ShapeDtypeStruct((32, ACCW), jnp.float32),
            jax.ShapeDtypeStruct((32, ACCW), jnp.float32),
        ),
        mesh=plsc.VectorSubcoreMesh(core_axis_name="c", subcore_axis_name="s"),
        scratch_types=[
            pltpu.VMEM((CH,), jnp.float32),
            pltpu.VMEM((CH,), jnp.float32),
            pltpu.VMEM((ACCW,), jnp.float32),
            pltpu.VMEM((ACCW,), jnp.float32),
            pltpu.SemaphoreType.DMA,
            pltpu.SemaphoreType.DMA,
        ],
    )
    up, low = sc(xf)

    out = pl.pallas_call(
        _finish_kernel,
        out_shape=jax.ShapeDtypeStruct((B, T), jnp.float32),
    )(up, low)
    return out


# speedup vs baseline: 1.5978x; 1.4433x over previous
import functools

import jax
import jax.numpy as jnp
from jax import lax
from jax.experimental import pallas as pl
from jax.experimental.pallas import tpu as pltpu
from jax.experimental.pallas import tpu_sc as plsc

T = 2048
T2 = T * T
STR = 8 * T
SPB = T // 8
SPT = 32
ACCW = 4096


def _sc_body(x_hbm, acc_hbm, bufA, bufB, acc, semA, semB):
    w = lax.axis_index("s") * 2 + lax.axis_index("c")
    b = w // 8
    s0 = (w % 8) * SPT
    base = b * T2
    lane = lax.iota(jnp.int32, 16)
    zero16 = jnp.zeros((16,), jnp.float32)
    colmask = jnp.where(lane < 15, 1.0, 0.0)

    def zbody(v, carry):
        acc[pl.ds(v * 16, 16)] = zero16
        return carry

    lax.fori_loop(0, ACCW // 16, zbody, 0)

    def start(buf, sem, s):
        pltpu.make_async_copy(
            x_hbm.at[pl.ds(base + s * STR, STR)], buf, sem
        ).start()

    def wait(buf, sem):
        pltpu.make_async_copy(x_hbm.at[pl.ds(0, STR)], buf, sem).wait()

    def compute_stripe(buf, s):
        r7 = jnp.where(s == SPB - 1, 0.0, 1.0)
        rb = (T - 1) - 8 * s
        for il in range(8):

            @plsc.parallel_loop(0, 127, unroll=4)
            def _(gx):
                o = (
                    lax.shift_left(lax.shift_right_logical(gx, 3), 10)
                    + lax.shift_left(jnp.bitwise_and(gx, 7), 4)
                    + il * 128
                )
                val = buf[pl.ds(o, 16)]
                if il == 7:
                    val = val * r7
                a0 = rb - il + gx * 16
                acc[pl.ds(a0, 16)] = acc[pl.ds(a0, 16)] + val

            val = buf[pl.ds(15 * 1024 + il * 128 + 7 * 16, 16)] * colmask
            if il == 7:
                val = val * r7
            a0 = rb - il + 127 * 16
            acc[pl.ds(a0, 16)] = acc[pl.ds(a0, 16)] + val

    start(bufA, semA, s0)

    def pair_body(p, carry):
        se = s0 + 2 * p
        so = se + 1
        wait(bufA, semA)
        start(bufB, semB, so)
        compute_stripe(bufA, se)
        wait(bufB, semB)

        @pl.when(p < SPT // 2 - 1)
        def _():
            start(bufA, semA, se + 2)

        compute_stripe(bufB, so)
        return carry

    lax.fori_loop(0, SPT // 2, pair_body, 0)

    pltpu.sync_copy(acc, acc_hbm.at[w])


def _finish_kernel(acc_ref, out_ref):
    rows = []
    for b in range(4):
        s = jnp.sum(acc_ref[8 * b : 8 * b + 8, :], axis=0, keepdims=True)
        rows.append(s[:, T // 2 - 1 : T // 2 - 1 + T])
    res = jnp.concatenate(rows, axis=0)
    jd = lax.broadcasted_iota(jnp.int32, (4, T), 1)
    count = (T - 1 - jnp.abs(jd - T // 2)).astype(jnp.float32)
    out_ref[...] = -res / count


@jax.jit
def kernel(x):
    B = x.shape[0]
    xt = (
        x.reshape(B, T // 8, 8, T // 128, 128)
        .swapaxes(2, 3)
        .reshape(-1)
    )

    sc = pl.kernel(
        _sc_body,
        out_type=jax.ShapeDtypeStruct((32, ACCW), jnp.float32),
        mesh=plsc.VectorSubcoreMesh(core_axis_name="c", subcore_axis_name="s"),
        scratch_types=[
            pltpu.VMEM((STR,), jnp.float32),
            pltpu.VMEM((STR,), jnp.float32),
            pltpu.VMEM((ACCW,), jnp.float32),
            pltpu.SemaphoreType.DMA,
            pltpu.SemaphoreType.DMA,
        ],
    )
    partial = sc(xt)

    out = pl.pallas_call(
        _finish_kernel,
        out_shape=jax.ShapeDtypeStruct((B, T), jnp.float32),
    )(partial)
    return out
